# baseline (speedup 1.0000x reference)
import functools

import jax
import jax.numpy as jnp
from jax import lax
from jax.experimental import pallas as pl


def _fp_kernel(a1_ref, n1_ref, f1_ref, a2_ref, n2_ref, f2_ref, w1a_ref,
               w1b_ref, b1_ref, w2_ref, b2_ref, out_ref, *, S):
    a1 = a1_ref[0]
    a2 = a2_ref[0]
    cross = jnp.dot(a1, a2.T, preferred_element_type=jnp.float32)
    d2 = n1_ref[0] + n2_ref[0] - 2.0 * cross
    work = jnp.maximum(d2, 0.0)

    tn = work.shape[0]
    acc = jnp.zeros((tn, S), jnp.float32)
    wsum = jnp.zeros((tn, 1), jnp.float32)
    cnt = jnp.zeros((tn, 1), jnp.float32)
    for _ in range(3):
        mk = jnp.min(work, axis=-1, keepdims=True)
        mask = work == mk
        c = jnp.sum(mask.astype(jnp.float32), axis=-1, keepdims=True)
        gate = cnt < 2.5
        wk = jnp.where(gate, 1.0 / jnp.maximum(jnp.sqrt(mk), 1e-10), 0.0)
        acc = acc + jnp.where(mask, wk, 0.0)
        wsum = wsum + wk * c
        cnt = cnt + c
        work = jnp.where(mask, jnp.inf, work)

    f2 = f2_ref[0]
    interp = jnp.dot(acc, f2, preferred_element_type=jnp.float32) / wsum
    f1 = f1_ref[0]
    h = jnp.dot(f1, w1a_ref[...], preferred_element_type=jnp.float32)
    h = h + jnp.dot(interp, w1b_ref[...], preferred_element_type=jnp.float32)
    h = jnp.maximum(h + b1_ref[...], 0.0)
    o = jnp.dot(h, w2_ref[...], preferred_element_type=jnp.float32)
    out_ref[0] = jnp.maximum(o + b2_ref[...], 0.0)


@jax.jit
def kernel(xyz1, xyz2, feat1, feat2, W1, b1, W2, b2):
    B, N, _ = xyz1.shape
    S = xyz2.shape[1]
    C1 = feat1.shape[-1]
    TN = 512

    n1 = jnp.sum(xyz1 * xyz1, axis=-1)[..., None]
    n2 = jnp.sum(xyz2 * xyz2, axis=-1)[:, None, :]
    a1 = jnp.concatenate([xyz1, jnp.zeros((B, N, 5), xyz1.dtype)], axis=-1)
    a2 = jnp.concatenate([xyz2, jnp.zeros((B, S, 5), xyz2.dtype)], axis=-1)

    W1a = W1[:C1]
    W1b = W1[C1:]
    b1r = b1.reshape(1, -1)
    b2r = b2.reshape(1, -1)

    grid = (B, N // TN)
    out = pl.pallas_call(
        functools.partial(_fp_kernel, S=S),
        grid=grid,
        in_specs=[
            pl.BlockSpec((1, TN, 8), lambda b, n: (b, n, 0)),
            pl.BlockSpec((1, TN, 1), lambda b, n: (b, n, 0)),
            pl.BlockSpec((1, TN, C1), lambda b, n: (b, n, 0)),
            pl.BlockSpec((1, S, 8), lambda b, n: (b, 0, 0)),
            pl.BlockSpec((1, 1, S), lambda b, n: (b, 0, 0)),
            pl.BlockSpec((1, S, feat2.shape[-1]), lambda b, n: (b, 0, 0)),
            pl.BlockSpec(W1a.shape, lambda b, n: (0, 0)),
            pl.BlockSpec(W1b.shape, lambda b, n: (0, 0)),
            pl.BlockSpec(b1r.shape, lambda b, n: (0, 0)),
            pl.BlockSpec(W2.shape, lambda b, n: (0, 0)),
            pl.BlockSpec(b2r.shape, lambda b, n: (0, 0)),
        ],
        out_specs=pl.BlockSpec((1, TN, W2.shape[-1]), lambda b, n: (b, n, 0)),
        out_shape=jax.ShapeDtypeStruct((B, N, W2.shape[-1]), jnp.float32),
    )(a1, n1, feat1, a2, n2, feat2, W1a, W1b, b1r, W2, b2r)
    return out


# speedup vs baseline: 1.1122x; 1.1122x over previous
import functools

import jax
import jax.numpy as jnp
from jax import lax
from jax.experimental import pallas as pl


def _fp_kernel(a1_ref, n1_ref, f1_ref, a2_ref, n2_ref, f2_ref, w1a_ref,
               w1b_ref, b1_ref, w2_ref, b2_ref, out_ref, *, S):
    a1 = a1_ref[0]
    a2 = a2_ref[0]
    cross = jnp.dot(a1, a2.T, preferred_element_type=jnp.float32)
    d2 = n1_ref[0] + n2_ref[0] - 2.0 * cross
    work = jnp.maximum(d2, 0.0)

    tn = work.shape[0]
    acc = jnp.zeros((tn, S), jnp.float32)
    wsum = jnp.zeros((tn, 1), jnp.float32)
    cnt = jnp.zeros((tn, 1), jnp.float32)
    for _ in range(3):
        mk = jnp.min(work, axis=-1, keepdims=True)
        mask = work == mk
        c = jnp.sum(mask.astype(jnp.float32), axis=-1, keepdims=True)
        gate = cnt < 2.5
        wk = jnp.where(gate, 1.0 / jnp.maximum(jnp.sqrt(mk), 1e-10), 0.0)
        acc = acc + jnp.where(mask, wk, 0.0)
        wsum = wsum + wk * c
        cnt = cnt + c
        work = jnp.where(mask, jnp.inf, work)

    f2 = f2_ref[0]
    interp = jnp.dot(acc, f2, preferred_element_type=jnp.float32) / wsum
    f1 = f1_ref[0]
    h = jnp.dot(f1, w1a_ref[...], preferred_element_type=jnp.float32)
    h = h + jnp.dot(interp, w1b_ref[...], preferred_element_type=jnp.float32)
    h = jnp.maximum(h + b1_ref[...], 0.0)
    o = jnp.dot(h, w2_ref[...], preferred_element_type=jnp.float32)
    out_ref[0] = jnp.maximum(o + b2_ref[...], 0.0)


@jax.jit
def kernel(xyz1, xyz2, feat1, feat2, W1, b1, W2, b2):
    B, N, _ = xyz1.shape
    S = xyz2.shape[1]
    C1 = feat1.shape[-1]
    TN = 1024

    n1 = jnp.sum(xyz1 * xyz1, axis=-1)[..., None]
    n2 = jnp.sum(xyz2 * xyz2, axis=-1)[:, None, :]
    a1 = jnp.concatenate([xyz1, jnp.zeros((B, N, 5), xyz1.dtype)], axis=-1)
    a2 = jnp.concatenate([xyz2, jnp.zeros((B, S, 5), xyz2.dtype)], axis=-1)

    W1a = W1[:C1]
    W1b = W1[C1:]
    b1r = b1.reshape(1, -1)
    b2r = b2.reshape(1, -1)

    grid = (B, N // TN)
    out = pl.pallas_call(
        functools.partial(_fp_kernel, S=S),
        grid=grid,
        in_specs=[
            pl.BlockSpec((1, TN, 8), lambda b, n: (b, n, 0)),
            pl.BlockSpec((1, TN, 1), lambda b, n: (b, n, 0)),
            pl.BlockSpec((1, TN, C1), lambda b, n: (b, n, 0)),
            pl.BlockSpec((1, S, 8), lambda b, n: (b, 0, 0)),
            pl.BlockSpec((1, 1, S), lambda b, n: (b, 0, 0)),
            pl.BlockSpec((1, S, feat2.shape[-1]), lambda b, n: (b, 0, 0)),
            pl.BlockSpec(W1a.shape, lambda b, n: (0, 0)),
            pl.BlockSpec(W1b.shape, lambda b, n: (0, 0)),
            pl.BlockSpec(b1r.shape, lambda b, n: (0, 0)),
            pl.BlockSpec(W2.shape, lambda b, n: (0, 0)),
            pl.BlockSpec(b2r.shape, lambda b, n: (0, 0)),
        ],
        out_specs=pl.BlockSpec((1, TN, W2.shape[-1]), lambda b, n: (b, n, 0)),
        out_shape=jax.ShapeDtypeStruct((B, N, W2.shape[-1]), jnp.float32),
    )(a1, n1, feat1, a2, n2, feat2, W1a, W1b, b1r, W2, b2r)
    return out


# speedup vs baseline: 1.1890x; 1.0691x over previous
import functools

import jax
import jax.numpy as jnp
from jax import lax
from jax.experimental import pallas as pl


def _fp_kernel(a1_ref, n1_ref, f1_ref, a2_ref, n2_ref, f2_ref, w1a_ref,
               w1b_ref, b1_ref, w2_ref, b2_ref, out_ref, *, S):
    a1 = a1_ref[0]
    a2 = a2_ref[0]
    cross = jnp.dot(a1, a2.T, preferred_element_type=jnp.float32)
    d2 = n1_ref[0] + n2_ref[0] - 2.0 * cross
    work = jnp.maximum(d2, 0.0)

    tn = work.shape[0]
    acc = jnp.zeros((tn, S), jnp.float32)
    wsum = jnp.zeros((tn, 1), jnp.float32)
    cnt = jnp.zeros((tn, 1), jnp.float32)
    for _ in range(3):
        mk = jnp.min(work, axis=-1, keepdims=True)
        mask = work == mk
        c = jnp.sum(mask.astype(jnp.float32), axis=-1, keepdims=True)
        gate = cnt < 2.5
        wk = jnp.where(gate, 1.0 / jnp.maximum(jnp.sqrt(mk), 1e-10), 0.0)
        acc = acc + jnp.where(mask, wk, 0.0)
        wsum = wsum + wk * c
        cnt = cnt + c
        work = jnp.where(mask, jnp.inf, work)

    f2 = f2_ref[0]
    interp = jnp.dot(acc, f2, preferred_element_type=jnp.float32) / wsum
    f1 = f1_ref[0]
    h = jnp.dot(f1, w1a_ref[...], preferred_element_type=jnp.float32)
    h = h + jnp.dot(interp, w1b_ref[...], preferred_element_type=jnp.float32)
    h = jnp.maximum(h + b1_ref[...], 0.0)
    o = jnp.dot(h, w2_ref[...], preferred_element_type=jnp.float32)
    out_ref[0] = jnp.maximum(o + b2_ref[...], 0.0)


@jax.jit
def kernel(xyz1, xyz2, feat1, feat2, W1, b1, W2, b2):
    B, N, _ = xyz1.shape
    S = xyz2.shape[1]
    C1 = feat1.shape[-1]
    TN = 2048

    n1 = jnp.sum(xyz1 * xyz1, axis=-1)[..., None]
    n2 = jnp.sum(xyz2 * xyz2, axis=-1)[:, None, :]
    a1 = jnp.concatenate([xyz1, jnp.zeros((B, N, 5), xyz1.dtype)], axis=-1)
    a2 = jnp.concatenate([xyz2, jnp.zeros((B, S, 5), xyz2.dtype)], axis=-1)

    W1a = W1[:C1]
    W1b = W1[C1:]
    b1r = b1.reshape(1, -1)
    b2r = b2.reshape(1, -1)

    grid = (B, N // TN)
    out = pl.pallas_call(
        functools.partial(_fp_kernel, S=S),
        grid=grid,
        in_specs=[
            pl.BlockSpec((1, TN, 8), lambda b, n: (b, n, 0)),
            pl.BlockSpec((1, TN, 1), lambda b, n: (b, n, 0)),
            pl.BlockSpec((1, TN, C1), lambda b, n: (b, n, 0)),
            pl.BlockSpec((1, S, 8), lambda b, n: (b, 0, 0)),
            pl.BlockSpec((1, 1, S), lambda b, n: (b, 0, 0)),
            pl.BlockSpec((1, S, feat2.shape[-1]), lambda b, n: (b, 0, 0)),
            pl.BlockSpec(W1a.shape, lambda b, n: (0, 0)),
            pl.BlockSpec(W1b.shape, lambda b, n: (0, 0)),
            pl.BlockSpec(b1r.shape, lambda b, n: (0, 0)),
            pl.BlockSpec(W2.shape, lambda b, n: (0, 0)),
            pl.BlockSpec(b2r.shape, lambda b, n: (0, 0)),
        ],
        out_specs=pl.BlockSpec((1, TN, W2.shape[-1]), lambda b, n: (b, n, 0)),
        out_shape=jax.ShapeDtypeStruct((B, N, W2.shape[-1]), jnp.float32),
    )(a1, n1, feat1, a2, n2, feat2, W1a, W1b, b1r, W2, b2r)
    return out


# speedup vs baseline: 1.2014x; 1.0103x over previous
import functools

import jax
import jax.numpy as jnp
from jax import lax
from jax.experimental import pallas as pl


def _fp_kernel(a1_ref, n1_ref, f1_ref, a2_ref, n2_ref, f2_ref, w1a_ref,
               w1b_ref, b1_ref, w2_ref, b2_ref, out_ref, *, S):
    a1 = a1_ref[0]
    a2 = a2_ref[0]
    cross = jnp.dot(a1, a2.T, preferred_element_type=jnp.float32)
    d2 = n1_ref[0] + n2_ref[0] - 2.0 * cross
    work = jnp.maximum(d2, 0.0)

    tn = work.shape[0]
    acc = jnp.zeros((tn, S), jnp.float32)
    wsum = jnp.zeros((tn, 1), jnp.float32)
    cnt = jnp.zeros((tn, 1), jnp.float32)
    for _ in range(3):
        mk = jnp.min(work, axis=-1, keepdims=True)
        mask = work == mk
        c = jnp.sum(mask.astype(jnp.float32), axis=-1, keepdims=True)
        gate = cnt < 2.5
        wk = jnp.where(gate, 1.0 / jnp.maximum(jnp.sqrt(mk), 1e-10), 0.0)
        acc = acc + jnp.where(mask, wk, 0.0)
        wsum = wsum + wk * c
        cnt = cnt + c
        work = jnp.where(mask, jnp.inf, work)

    f2 = f2_ref[0]
    interp = jnp.dot(acc, f2, preferred_element_type=jnp.float32) / wsum
    f1 = f1_ref[0]
    h = jnp.dot(f1, w1a_ref[...], preferred_element_type=jnp.float32)
    h = h + jnp.dot(interp, w1b_ref[...], preferred_element_type=jnp.float32)
    h = jnp.maximum(h + b1_ref[...], 0.0)
    o = jnp.dot(h, w2_ref[...], preferred_element_type=jnp.float32)
    out_ref[0] = jnp.maximum(o + b2_ref[...], 0.0)


@jax.jit
def kernel(xyz1, xyz2, feat1, feat2, W1, b1, W2, b2):
    B, N, _ = xyz1.shape
    S = xyz2.shape[1]
    C1 = feat1.shape[-1]
    TN = 4096

    n1 = jnp.sum(xyz1 * xyz1, axis=-1)[..., None]
    n2 = jnp.sum(xyz2 * xyz2, axis=-1)[:, None, :]
    a1 = jnp.concatenate([xyz1, jnp.zeros((B, N, 5), xyz1.dtype)], axis=-1)
    a2 = jnp.concatenate([xyz2, jnp.zeros((B, S, 5), xyz2.dtype)], axis=-1)

    W1a = W1[:C1]
    W1b = W1[C1:]
    b1r = b1.reshape(1, -1)
    b2r = b2.reshape(1, -1)

    grid = (B, N // TN)
    out = pl.pallas_call(
        functools.partial(_fp_kernel, S=S),
        grid=grid,
        in_specs=[
            pl.BlockSpec((1, TN, 8), lambda b, n: (b, n, 0)),
            pl.BlockSpec((1, TN, 1), lambda b, n: (b, n, 0)),
            pl.BlockSpec((1, TN, C1), lambda b, n: (b, n, 0)),
            pl.BlockSpec((1, S, 8), lambda b, n: (b, 0, 0)),
            pl.BlockSpec((1, 1, S), lambda b, n: (b, 0, 0)),
            pl.BlockSpec((1, S, feat2.shape[-1]), lambda b, n: (b, 0, 0)),
            pl.BlockSpec(W1a.shape, lambda b, n: (0, 0)),
            pl.BlockSpec(W1b.shape, lambda b, n: (0, 0)),
            pl.BlockSpec(b1r.shape, lambda b, n: (0, 0)),
            pl.BlockSpec(W2.shape, lambda b, n: (0, 0)),
            pl.BlockSpec(b2r.shape, lambda b, n: (0, 0)),
        ],
        out_specs=pl.BlockSpec((1, TN, W2.shape[-1]), lambda b, n: (b, n, 0)),
        out_shape=jax.ShapeDtypeStruct((B, N, W2.shape[-1]), jnp.float32),
    )(a1, n1, feat1, a2, n2, feat2, W1a, W1b, b1r, W2, b2r)
    return out


# speedup vs baseline: 1.2419x; 1.0338x over previous
import functools

import jax
import jax.numpy as jnp
from jax import lax
from jax.experimental import pallas as pl


def _fp_kernel(a1_ref, n1_ref, f1_ref, a2_ref, n2_ref, f2_ref, w1a_ref,
               w1b_ref, b1_ref, w2_ref, b2_ref, out_ref, *, S):
    a1 = a1_ref[0]
    a2 = a2_ref[0]
    cross = jnp.dot(a1, a2.T, preferred_element_type=jnp.float32)
    work = (n1_ref[0] + n2_ref[0]) - 2.0 * cross

    tn = work.shape[0]
    acc = jnp.zeros((tn, S), jnp.float32)
    wsum = jnp.zeros((tn, 1), jnp.float32)
    cnt = jnp.zeros((tn, 1), jnp.float32)
    for k in range(3):
        mk = jnp.min(work, axis=-1, keepdims=True)
        mask = work == mk
        m01 = jnp.where(mask, 1.0, 0.0)
        c = jnp.sum(m01, axis=-1, keepdims=True)
        gate = cnt < 2.5
        dk = jnp.maximum(jnp.sqrt(jnp.maximum(mk, 0.0)), 1e-10)
        wk = jnp.where(gate, 1.0 / dk, 0.0)
        acc = acc + m01 * wk
        wsum = wsum + wk * c
        cnt = cnt + c
        if k < 2:
            work = jnp.where(mask, jnp.inf, work)

    f2 = f2_ref[0]
    interp = jnp.dot(acc, f2, preferred_element_type=jnp.float32) / wsum
    f1 = f1_ref[0]
    h = jnp.dot(f1, w1a_ref[...], preferred_element_type=jnp.float32)
    h = h + jnp.dot(interp, w1b_ref[...], preferred_element_type=jnp.float32)
    h = jnp.maximum(h + b1_ref[...], 0.0)
    o = jnp.dot(h, w2_ref[...], preferred_element_type=jnp.float32)
    out_ref[0] = jnp.maximum(o + b2_ref[...], 0.0)


@jax.jit
def kernel(xyz1, xyz2, feat1, feat2, W1, b1, W2, b2):
    B, N, _ = xyz1.shape
    S = xyz2.shape[1]
    C1 = feat1.shape[-1]
    TN = 4096

    n1 = jnp.sum(xyz1 * xyz1, axis=-1)[..., None]
    n2 = jnp.sum(xyz2 * xyz2, axis=-1)[:, None, :]
    a1 = jnp.concatenate([xyz1, jnp.zeros((B, N, 5), xyz1.dtype)], axis=-1)
    a2 = jnp.concatenate([xyz2, jnp.zeros((B, S, 5), xyz2.dtype)], axis=-1)

    W1a = W1[:C1]
    W1b = W1[C1:]
    b1r = b1.reshape(1, -1)
    b2r = b2.reshape(1, -1)

    grid = (B, N // TN)
    out = pl.pallas_call(
        functools.partial(_fp_kernel, S=S),
        grid=grid,
        in_specs=[
            pl.BlockSpec((1, TN, 8), lambda b, n: (b, n, 0)),
            pl.BlockSpec((1, TN, 1), lambda b, n: (b, n, 0)),
            pl.BlockSpec((1, TN, C1), lambda b, n: (b, n, 0)),
            pl.BlockSpec((1, S, 8), lambda b, n: (b, 0, 0)),
            pl.BlockSpec((1, 1, S), lambda b, n: (b, 0, 0)),
            pl.BlockSpec((1, S, feat2.shape[-1]), lambda b, n: (b, 0, 0)),
            pl.BlockSpec(W1a.shape, lambda b, n: (0, 0)),
            pl.BlockSpec(W1b.shape, lambda b, n: (0, 0)),
            pl.BlockSpec(b1r.shape, lambda b, n: (0, 0)),
            pl.BlockSpec(W2.shape, lambda b, n: (0, 0)),
            pl.BlockSpec(b2r.shape, lambda b, n: (0, 0)),
        ],
        out_specs=pl.BlockSpec((1, TN, W2.shape[-1]), lambda b, n: (b, n, 0)),
        out_shape=jax.ShapeDtypeStruct((B, N, W2.shape[-1]), jnp.float32),
    )(a1, n1, feat1, a2, n2, feat2, W1a, W1b, b1r, W2, b2r)
    return out
